# Initial kernel scaffold; baseline (speedup 1.0000x reference)
import jax
import jax.numpy as jnp
from jax.experimental import pallas as pl


def kernel(x, edge_index, h_prev, weight, bias, W_ih, W_hh, b_ih, b_hh):
    raise NotImplementedError("write your pallas kernel here")


# speedup vs baseline: 3.5147x; 3.5147x over previous
import dataclasses
import functools

import jax
import jax.numpy as jnp
from jax import lax
from jax.experimental import pallas as pl
from jax.experimental.pallas import tpu as pltpu
from jax.experimental.pallas import tpu_sc as plsc

N = 10000
E = 160000
E2 = 2 * E
D = 256
H = 128
NC, NS = 2, 16
NB = 160
CB = 16
EPAD = NS * NB * 128
ACC_ROWS = 10112
HIST = 10240
BN = 1000

_sc_mesh = plsc.VectorSubcoreMesh(core_axis_name="c", subcore_axis_name="s")

_sc_params = pltpu.CompilerParams()
if "needs_layout_passes" in pltpu.CompilerParams.__dataclass_fields__:
    _sc_params = dataclasses.replace(_sc_params, needs_layout_passes=False)


@functools.partial(
    pl.kernel,
    out_type=jax.ShapeDtypeStruct((NC * NS, HIST), jnp.float32),
    mesh=_sc_mesh,
    scratch_types=[
        pltpu.VMEM((E2 // (NC * NS),), jnp.int32),
        pltpu.VMEM((HIST,), jnp.float32),
        pltpu.SemaphoreType.DMA,
    ],
    compiler_params=_sc_params,
)
def _deg_kernel(ends_hbm, hist_hbm, idx_v, hist_v, sem):
    chunk = E2 // (NC * NS)
    wid = lax.axis_index("s") * NC + lax.axis_index("c")
    pltpu.async_copy(ends_hbm.at[pl.ds(wid * chunk, chunk)], idx_v, sem).wait()

    @pl.loop(0, HIST, step=16)
    def _(i):
        hist_v[pl.ds(i, 16)] = jnp.zeros((16,), jnp.float32)

    ones = jnp.ones((16,), jnp.float32)

    @pl.loop(0, chunk, step=16)
    def _(i):
        plsc.addupdate_scatter(hist_v, [idx_v[pl.ds(i, 16)]], ones)

    pltpu.async_copy(hist_v, hist_hbm.at[wid], sem).wait()


def _k0_body(hist_ref, dinv_ref):
    deg = jnp.sum(hist_ref[...], axis=0)
    dinv_ref[...] = jnp.where(deg > 0.0, lax.rsqrt(deg), 0.0)


_k0_call = pl.pallas_call(
    _k0_body,
    in_specs=[pl.BlockSpec((NC * NS, HIST), lambda: (0, 0))],
    out_specs=pl.BlockSpec((HIST,), lambda: (0,)),
    out_shape=jax.ShapeDtypeStruct((HIST,), jnp.float32),
)


def _b_body(x_ref, w_ref, b_ref, dinv_ref, y_ref):
    x1 = jnp.dot(x_ref[...], w_ref[...], preferred_element_type=jnp.float32)
    x1 = jnp.maximum(x1 + b_ref[...], 0.0)
    y_ref[...] = x1 * dinv_ref[...]


_b_call = pl.pallas_call(
    _b_body,
    grid=(2, N // BN),
    in_specs=[
        pl.BlockSpec((BN, D), lambda c, i: (i, 0)),
        pl.BlockSpec((D, H), lambda c, i: (0, c)),
        pl.BlockSpec((1, H), lambda c, i: (0, c)),
        pl.BlockSpec((BN, 1), lambda c, i: (i, 0)),
    ],
    out_specs=pl.BlockSpec((BN, H), lambda c, i: (c * (N // BN) + i, 0)),
    out_shape=jax.ShapeDtypeStruct((2 * N, H), jnp.float32),
)


@functools.partial(
    pl.kernel,
    out_type=jax.ShapeDtypeStruct((NC, ACC_ROWS, H), jnp.float32),
    mesh=_sc_mesh,
    scratch_types=[
        pltpu.VMEM_SHARED((ACC_ROWS, H), jnp.float32),
        pltpu.VMEM((CB, 128), jnp.int32),
        pltpu.VMEM((CB, 128), jnp.int32),
        pltpu.VMEM((128, H), jnp.float32),
        pltpu.SemaphoreType.DMA,
        pltpu.SemaphoreType.DMA,
    ],
)
def _gs_kernel(y_hbm, ridx_hbm, cidx_hbm, z_hbm, out_hbm,
               acc_s, ridx_v, cidx_v, rows_v, sem, gsem):
    cid = lax.axis_index("c")
    sid = lax.axis_index("s")
    zrows = ACC_ROWS // NS
    pltpu.async_copy(z_hbm.at[pl.ds(sid * zrows, zrows)],
                     acc_s.at[pl.ds(sid * zrows, zrows)], sem).wait()
    plsc.subcore_barrier()

    @pl.loop(0, NB // CB)
    def _(c):
        pltpu.async_copy(ridx_hbm.at[sid, pl.ds(c * CB, CB)], ridx_v, sem).wait()
        pltpu.async_copy(cidx_hbm.at[cid, sid, pl.ds(c * CB, CB)], cidx_v, sem).wait()

        @pl.loop(0, CB)
        def _(b):
            pltpu.async_copy(y_hbm.at[cidx_v.at[b]], rows_v, gsem).wait()
            pltpu.sync_copy(rows_v, acc_s.at[ridx_v.at[b]], add=True)

    plsc.subcore_barrier()
    pltpu.async_copy(acc_s.at[pl.ds(sid * zrows, zrows)],
                     out_hbm.at[cid, pl.ds(sid * zrows, zrows)], sem).wait()


def _d_body(o_ref, dinv_ref, h_ref, wih_ref, whh_ref, bih_ref, bhh_ref,
            out_ref, hnew_ref):
    out_blk = jnp.concatenate([o_ref[0], o_ref[1]], axis=1) * dinv_ref[...]
    h = h_ref[...]
    gi = jnp.dot(out_blk, wih_ref[...], preferred_element_type=jnp.float32)
    gi = gi + bih_ref[...]
    gh = jnp.dot(h, whh_ref[...], preferred_element_type=jnp.float32)
    gh = gh + bhh_ref[...]
    r = jax.nn.sigmoid(gi[:, :D] + gh[:, :D])
    z = jax.nn.sigmoid(gi[:, D:2 * D] + gh[:, D:2 * D])
    n = jnp.tanh(gi[:, 2 * D:] + r * gh[:, 2 * D:])
    out_ref[...] = out_blk
    hnew_ref[...] = (1.0 - z) * n + z * h


_d_call = pl.pallas_call(
    _d_body,
    grid=(N // BN,),
    in_specs=[
        pl.BlockSpec((NC, BN, H), lambda i: (0, i, 0)),
        pl.BlockSpec((BN, 1), lambda i: (i, 0)),
        pl.BlockSpec((BN, D), lambda i: (i, 0)),
        pl.BlockSpec((D, 3 * D), lambda i: (0, 0)),
        pl.BlockSpec((D, 3 * D), lambda i: (0, 0)),
        pl.BlockSpec((1, 3 * D), lambda i: (0, 0)),
        pl.BlockSpec((1, 3 * D), lambda i: (0, 0)),
    ],
    out_specs=[
        pl.BlockSpec((BN, D), lambda i: (i, 0)),
        pl.BlockSpec((BN, D), lambda i: (i, 0)),
    ],
    out_shape=[
        jax.ShapeDtypeStruct((N, D), jnp.float32),
        jax.ShapeDtypeStruct((N, D), jnp.float32),
    ],
)


def kernel(x, edge_index, h_prev, weight, bias, W_ih, W_hh, b_ih, b_hh):
    ei = edge_index.astype(jnp.int32)

    row = jnp.concatenate([ei[0], ei[1]])
    col = jnp.concatenate([ei[1], ei[0]])

    hist = _deg_kernel(row)
    dinv_lane = _k0_call(hist)
    dinv = dinv_lane[:N].reshape(N, 1)
    y_tab = _b_call(x, weight, bias.reshape(1, D), dinv)
    npad = EPAD - E2
    rowp = jnp.concatenate([row, jnp.full((npad,), N, jnp.int32)])
    colp = jnp.concatenate([col, jnp.zeros((npad,), jnp.int32)])
    ridx3 = rowp.reshape(NS, NB, 128)
    cidx4 = jnp.stack([colp, colp + N]).reshape(NC, NS, NB, 128)
    zeros = jnp.zeros((ACC_ROWS, H), jnp.float32)

    out_raw = _gs_kernel(y_tab, ridx3, cidx4, zeros)
    out, h_new = _d_call(out_raw, dinv, h_prev,
                         W_ih.T, W_hh.T,
                         b_ih.reshape(1, 3 * D), b_hh.reshape(1, 3 * D))
    return (out, h_new)
